# baseline (speedup 1.0000x reference)
import functools

import jax
import jax.numpy as jnp
from jax import lax
from jax.experimental import pallas as pl
from jax.experimental.pallas import tpu as pltpu
from jax.experimental.pallas import tpu_sc as plsc

B = 16384
NCAT = 26
NNUM = 13
VOCAB = 100000
EMB = 4
DEEP_IN = EMB * NCAT + NNUM

NC = 2
NS = 16
NW = NC * NS
N_IDX = B * NCAT
N_PER_W = N_IDX // NW
CH = 128
HALVES = 2
W_HALF = N_PER_W // HALVES
D_HALF = W_HALF * EMB
WCH = W_HALF // CH
DCH = D_HALF // CH

BM = 1024


def _sc_deep_body(deep_tab, idxb_hbm, deep_out,
                  idxb_v, idxe_v, deep_v, sem_d):
    c = lax.axis_index("c")
    s = lax.axis_index("s")
    wid = s * NC + c
    lane = lax.iota(jnp.int32, 16)
    c4 = lax.shift_right_logical(lane, 2)
    m4 = lax.bitwise_and(lane, 3) * VOCAB

    pltpu.sync_copy(idxb_hbm.at[wid], idxb_v)

    def expand(g, cy):
        base = plsc.load_gather(idxb_v, [4 * g + c4])
        idxe_v[pl.ds(g * 16, 16)] = base + m4
        return cy

    lax.fori_loop(0, N_PER_W * EMB // 16, expand, 0)

    def fire_d(j, cy):
        pltpu.async_copy(deep_tab.at[idxe_v.at[pl.ds(j * CH, CH)]],
                         deep_v.at[pl.ds(j * CH, CH)], sem_d)
        return cy

    lax.fori_loop(0, N_PER_W * EMB // CH, fire_d, 0)
    pltpu.make_async_copy(deep_out.at[wid], deep_v, sem_d).wait()
    pltpu.sync_copy(deep_v, deep_out.at[wid])


def _sc_wide_body(wide_tab, idxw_hbm, wide_out, idxw_v, wide_v, sem_w):
    c = lax.axis_index("c")
    s = lax.axis_index("s")
    wid = s * NC + c
    pltpu.sync_copy(idxw_hbm.at[wid], idxw_v)

    def fire_w(j, cy):
        pltpu.async_copy(wide_tab.at[idxw_v.at[pl.ds(j * CH, CH)]],
                         wide_v.at[pl.ds(j * CH, CH)], sem_w)
        return cy

    lax.fori_loop(0, N_PER_W // CH, fire_w, 0)
    pltpu.make_async_copy(wide_out.at[wid], wide_v, sem_w).wait()
    pltpu.sync_copy(wide_v, wide_out.at[wid])


def _sc_mesh():
    return plsc.VectorSubcoreMesh(core_axis_name="c", subcore_axis_name="s",
                                  num_cores=NC, num_subcores=NS)


def _sc_deep(deep_tab, idxb):
    f = pl.kernel(
        _sc_deep_body,
        out_type=jax.ShapeDtypeStruct((NW, N_PER_W * EMB), jnp.float32),
        mesh=_sc_mesh(),
        scratch_types=[
            pltpu.VMEM((N_PER_W,), jnp.int32),
            pltpu.VMEM((N_PER_W * EMB,), jnp.int32),
            pltpu.VMEM((N_PER_W * EMB,), jnp.float32),
            pltpu.SemaphoreType.DMA,
        ],
        compiler_params=pltpu.CompilerParams(use_tc_tiling_on_sc=False,
                                             needs_layout_passes=False),
    )
    return f(deep_tab, idxb)


def _sc_wide(wide_tab, idxw):
    f = pl.kernel(
        _sc_wide_body,
        out_type=jax.ShapeDtypeStruct((NW, N_PER_W), jnp.float32),
        mesh=_sc_mesh(),
        scratch_types=[
            pltpu.VMEM((N_PER_W,), jnp.int32),
            pltpu.VMEM((N_PER_W,), jnp.float32),
            pltpu.SemaphoreType.DMA,
        ],
        compiler_params=pltpu.CompilerParams(use_tc_tiling_on_sc=False,
                                             needs_layout_passes=False),
    )
    return f(wide_tab, idxw)


def _mlp_body(num_ref, deepg_ref, w1_ref, b1_ref,
              w2_ref, b2_ref, w3_ref, b3_ref, wd_ref, bd_ref, d_ref):
    bf = jnp.bfloat16
    f32 = jnp.float32

    def bdot(a, w):
        return jax.lax.dot_general(a.astype(bf), w.astype(bf),
                                   (((1,), (0,)), ((), ())),
                                   preferred_element_type=f32)

    deep_in = jnp.concatenate([num_ref[...], deepg_ref[...]], axis=1)
    h = jnp.maximum(bdot(deep_in, w1_ref[...]) + b1_ref[...], 0.0)
    h = jnp.maximum(bdot(h, w2_ref[...]) + b2_ref[...], 0.0)
    h = jnp.maximum(bdot(h, w3_ref[...]) + b3_ref[...], 0.0)
    d_ref[...] = bdot(h, wd_ref[...]) + bd_ref[0, 0]


def _combine_body(d_ref, wide_ref, wf_ref, bf_ref, out_ref):
    bf = jnp.bfloat16
    f32 = jnp.float32

    def rnd(v):
        return v.astype(bf).astype(f32)

    w = jnp.sum(wide_ref[...], axis=1, keepdims=True)
    w = w + w
    wb, db = rnd(w), rnd(d_ref[...])
    o0 = wb * rnd(wf_ref[0, 0]) + db * rnd(wf_ref[0, 1]) + bf_ref[0, 0]
    o1 = wb * rnd(wf_ref[1, 0]) + db * rnd(wf_ref[1, 1]) + bf_ref[0, 1]
    z = jnp.concatenate([o0, o1], axis=1)
    out_ref[...] = jax.nn.sigmoid(z)


def _mlp_deep(num_x, deepg, W1, b1, W2, b2, W3, b3, Wd, bd):
    grid = (B // BM,)
    row = lambda i: (i, 0)
    whole = lambda i: (0, 0)
    smem = functools.partial(pl.BlockSpec, memory_space=pltpu.SMEM)
    return pl.pallas_call(
        _mlp_body,
        grid=grid,
        in_specs=[
            pl.BlockSpec((BM, NNUM), row),
            pl.BlockSpec((BM, EMB * NCAT), row),
            pl.BlockSpec((DEEP_IN, 1024), whole),
            pl.BlockSpec((1, 1024), whole),
            pl.BlockSpec((1024, 512), whole),
            pl.BlockSpec((1, 512), whole),
            pl.BlockSpec((512, 256), whole),
            pl.BlockSpec((1, 256), whole),
            pl.BlockSpec((256, 1), whole),
            smem((1, 1), whole),
        ],
        out_specs=pl.BlockSpec((BM, 1), row),
        out_shape=jax.ShapeDtypeStruct((B, 1), jnp.float32),
        compiler_params=pltpu.CompilerParams(
            dimension_semantics=("parallel",)),
    )(num_x, deepg, W1, b1, W2, b2, W3, b3, Wd, bd)


def _combine(d, wide, Wf, bfv):
    grid = (B // BM,)
    row = lambda i: (i, 0)
    whole = lambda i: (0, 0)
    smem = functools.partial(pl.BlockSpec, memory_space=pltpu.SMEM)
    return pl.pallas_call(
        _combine_body,
        grid=grid,
        in_specs=[
            pl.BlockSpec((BM, 1), row),
            pl.BlockSpec((BM, NCAT), row),
            smem((2, 2), whole),
            smem((1, 2), whole),
        ],
        out_specs=pl.BlockSpec((BM, 2), row),
        out_shape=jax.ShapeDtypeStruct((B, 2), jnp.float32),
        compiler_params=pltpu.CompilerParams(
            dimension_semantics=("parallel",)),
    )(d, wide, Wf, bfv)


def kernel(x, wide_emb, deep_emb, Wn, bn, W1, b1, W2, b2, W3, b3, Wd, bd, Wf, bf):
    idx0 = x[:, :NCAT].astype(jnp.int32)
    idxw = (idx0 + jnp.arange(NCAT, dtype=jnp.int32) * VOCAB
            ).reshape(NW, N_PER_W)
    idxb = (idx0 + jnp.arange(NCAT, dtype=jnp.int32) * (EMB * VOCAB)
            ).reshape(NW, N_PER_W)
    deep_tab = deep_emb.transpose(0, 2, 1).reshape(NCAT * VOCAB * EMB)
    wide_tab = wide_emb.reshape(NCAT * VOCAB)

    deep_g = _sc_deep(deep_tab, idxb)
    wide_g = _sc_wide(wide_tab, idxw)
    deepg = deep_g.reshape(B, NCAT * EMB)
    wide = wide_g.reshape(B, NCAT)

    num_x = x[:, NCAT:]
    d = _mlp_deep(num_x, deepg,
                  W1, b1.reshape(1, 1024),
                  W2, b2.reshape(1, 512), W3, b3.reshape(1, 256),
                  Wd, bd.reshape(1, 1))
    return _combine(d, wide, Wf, bf.reshape(1, 2))


# speedup vs baseline: 1.0465x; 1.0465x over previous
import functools

import jax
import jax.numpy as jnp
from jax import lax
from jax.experimental import pallas as pl
from jax.experimental.pallas import tpu as pltpu
from jax.experimental.pallas import tpu_sc as plsc

B = 16384
NCAT = 26
NNUM = 13
VOCAB = 100000
EMB = 4
DEEP_IN = EMB * NCAT + NNUM

NC = 2
NS = 16
NW = NC * NS
N_IDX = B * NCAT
N_PER_W = N_IDX // NW
CH = 128
HALVES = 2
W_HALF = N_PER_W // HALVES
D_HALF = W_HALF * EMB
WCH = W_HALF // CH
DCH = D_HALF // CH

BM = 1024


def _sc_deep_body(deep_tab, idxb_hbm, deep_out,
                  idxb_v, idxe_v, deep_v, sem_d):
    c = lax.axis_index("c")
    s = lax.axis_index("s")
    wid = s * NC + c
    lane = lax.iota(jnp.int32, 16)
    c4 = lax.shift_right_logical(lane, 2)
    m4 = lax.bitwise_and(lane, 3) * VOCAB

    pltpu.sync_copy(idxb_hbm.at[wid], idxb_v)

    def expand(g, cy):
        base = plsc.load_gather(idxb_v, [4 * g + c4])
        idxe_v[pl.ds(g * 16, 16)] = base + m4
        return cy

    lax.fori_loop(0, N_PER_W * EMB // 16, expand, 0)

    def fire_d(j, cy):
        pltpu.async_copy(deep_tab.at[idxe_v.at[pl.ds(j * CH, CH)]],
                         deep_v.at[pl.ds(j * CH, CH)], sem_d)
        return cy

    lax.fori_loop(0, N_PER_W * EMB // CH, fire_d, 0)
    pltpu.make_async_copy(deep_out.at[wid], deep_v, sem_d).wait()
    pltpu.sync_copy(deep_v, deep_out.at[wid])


def _sc_wide_body(wide_tab, idxw_hbm, wide_out, idxw_v, wide_v, sem_w):
    c = lax.axis_index("c")
    s = lax.axis_index("s")
    wid = s * NC + c
    pltpu.sync_copy(idxw_hbm.at[wid], idxw_v)

    def fire_w(j, cy):
        pltpu.async_copy(wide_tab.at[idxw_v.at[pl.ds(j * CH, CH)]],
                         wide_v.at[pl.ds(j * CH, CH)], sem_w)
        return cy

    lax.fori_loop(0, N_PER_W // CH, fire_w, 0)
    pltpu.make_async_copy(wide_out.at[wid], wide_v, sem_w).wait()
    pltpu.sync_copy(wide_v, wide_out.at[wid])


def _sc_mesh():
    return plsc.VectorSubcoreMesh(core_axis_name="c", subcore_axis_name="s",
                                  num_cores=NC, num_subcores=NS)


def _sc_deep(deep_tab, idxb):
    f = pl.kernel(
        _sc_deep_body,
        out_type=jax.ShapeDtypeStruct((NW, N_PER_W * EMB), jnp.float32),
        mesh=_sc_mesh(),
        scratch_types=[
            pltpu.VMEM((N_PER_W,), jnp.int32),
            pltpu.VMEM((N_PER_W * EMB,), jnp.int32),
            pltpu.VMEM((N_PER_W * EMB,), jnp.float32),
            pltpu.SemaphoreType.DMA,
        ],
        compiler_params=pltpu.CompilerParams(use_tc_tiling_on_sc=False,
                                             needs_layout_passes=False),
    )
    return f(deep_tab, idxb)


def _sc_wide(wide_tab, idxw):
    f = pl.kernel(
        _sc_wide_body,
        out_type=jax.ShapeDtypeStruct((NW, N_PER_W), jnp.float32),
        mesh=_sc_mesh(),
        scratch_types=[
            pltpu.VMEM((N_PER_W,), jnp.int32),
            pltpu.VMEM((N_PER_W,), jnp.float32),
            pltpu.SemaphoreType.DMA,
        ],
        compiler_params=pltpu.CompilerParams(use_tc_tiling_on_sc=False,
                                             needs_layout_passes=False),
    )
    return f(wide_tab, idxw)


def _mlp_body(num_ref, deepg_ref, wide_ref, w1_ref, b1_ref,
              w2_ref, b2_ref, w3_ref, b3_ref, wd_ref, bd_ref,
              wf_ref, bf_ref, out_ref):
    bf = jnp.bfloat16
    f32 = jnp.float32

    def bdot(a, w):
        return jax.lax.dot_general(a.astype(bf), w.astype(bf),
                                   (((1,), (0,)), ((), ())),
                                   preferred_element_type=f32)

    def rnd(v):
        return v.astype(bf).astype(f32)

    deep_in = jnp.concatenate([num_ref[...], deepg_ref[...]], axis=1)
    h = jnp.maximum(bdot(deep_in, w1_ref[...]) + b1_ref[...], 0.0)
    h = jnp.maximum(bdot(h, w2_ref[...]) + b2_ref[...], 0.0)
    h = jnp.maximum(bdot(h, w3_ref[...]) + b3_ref[...], 0.0)
    d = bdot(h, wd_ref[...]) + bd_ref[0, 0]
    w = jnp.sum(wide_ref[...], axis=1, keepdims=True)
    w = w + w
    wb, db = rnd(w), rnd(d)
    o0 = wb * rnd(wf_ref[0, 0]) + db * rnd(wf_ref[0, 1]) + bf_ref[0, 0]
    o1 = wb * rnd(wf_ref[1, 0]) + db * rnd(wf_ref[1, 1]) + bf_ref[0, 1]
    z = jnp.concatenate([o0, o1], axis=1)
    out_ref[...] = jax.nn.sigmoid(z)


def _mlp(num_x, deepg, wide, W1, b1, W2, b2, W3, b3, Wd, bd, Wf, bf):
    grid = (B // BM,)
    row = lambda i: (i, 0)
    whole = lambda i: (0, 0)
    smem = functools.partial(pl.BlockSpec, memory_space=pltpu.SMEM)
    return pl.pallas_call(
        _mlp_body,
        grid=grid,
        in_specs=[
            pl.BlockSpec((BM, NNUM), row),
            pl.BlockSpec((BM, EMB * NCAT), row),
            pl.BlockSpec((BM, NCAT), row),
            pl.BlockSpec((DEEP_IN, 1024), whole),
            pl.BlockSpec((1, 1024), whole),
            pl.BlockSpec((1024, 512), whole),
            pl.BlockSpec((1, 512), whole),
            pl.BlockSpec((512, 256), whole),
            pl.BlockSpec((1, 256), whole),
            pl.BlockSpec((256, 1), whole),
            smem((1, 1), whole),
            smem((2, 2), whole),
            smem((1, 2), whole),
        ],
        out_specs=pl.BlockSpec((BM, 2), row),
        out_shape=jax.ShapeDtypeStruct((B, 2), jnp.float32),
        compiler_params=pltpu.CompilerParams(
            dimension_semantics=("parallel",)),
    )(num_x, deepg, wide, W1, b1, W2, b2, W3, b3, Wd, bd, Wf, bf)


def kernel(x, wide_emb, deep_emb, Wn, bn, W1, b1, W2, b2, W3, b3, Wd, bd, Wf, bf):
    idx0 = x[:, :NCAT].astype(jnp.int32)
    idxw = (idx0 + jnp.arange(NCAT, dtype=jnp.int32) * VOCAB
            ).reshape(NW, N_PER_W)
    idxb = (idx0 + jnp.arange(NCAT, dtype=jnp.int32) * (EMB * VOCAB)
            ).reshape(NW, N_PER_W)
    deep_tab = deep_emb.transpose(0, 2, 1).reshape(NCAT * VOCAB * EMB)
    wide_tab = wide_emb.reshape(NCAT * VOCAB)

    deep_g = _sc_deep(deep_tab, idxb)
    wide_g = _sc_wide(wide_tab, idxw)
    deepg = deep_g.reshape(B, NCAT * EMB)
    wide = wide_g.reshape(B, NCAT)

    num_x = x[:, NCAT:]
    return _mlp(num_x, deepg, wide,
                W1, b1.reshape(1, 1024),
                W2, b2.reshape(1, 512), W3, b3.reshape(1, 256),
                Wd, bd.reshape(1, 1), Wf, bf.reshape(1, 2))
